# Initial kernel scaffold; baseline (speedup 1.0000x reference)
import jax
import jax.numpy as jnp
from jax.experimental import pallas as pl


def kernel(z_number, nbrs, dist, atom_table, W_dist, W_cat, b_cat):
    raise NotImplementedError("write your pallas kernel here")


# speedup vs baseline: 12.5893x; 12.5893x over previous
import jax
import jax.numpy as jnp
from jax import lax
from jax.experimental import pallas as pl
from jax.experimental.pallas import tpu as pltpu
from jax.experimental.pallas import tpu_sc as plsc

N_ATOMS = 10000
N_EDGES = 320000
FEAT = 128
N_RBF = 64
CUTOFF = 5.0

NUM_SC = 2
NUM_SUBCORES = 16
NUM_TILES = NUM_SC * NUM_SUBCORES
EDGES_PER_TILE = N_EDGES // NUM_TILES
N_PAD = 10240
ROWS_PER_SUBCORE = N_PAD // NUM_SUBCORES
EDGE_WIN = 128

EBL = 2560
NEB = N_EDGES // EBL
NB = 2000

_sc_mesh = plsc.VectorSubcoreMesh(core_axis_name="c", subcore_axis_name="s")


def _zc_gather(z1d, src1d):
    @pl.kernel(
        out_type=jax.ShapeDtypeStruct((N_EDGES,), jnp.int32),
        mesh=_sc_mesh,
        compiler_params=pltpu.CompilerParams(needs_layout_passes=False),
        scratch_types=[
            pltpu.VMEM((N_ATOMS,), jnp.int32),
            pltpu.VMEM((EDGES_PER_TILE,), jnp.int32),
            pltpu.VMEM((EDGES_PER_TILE,), jnp.int32),
        ],
    )
    def k(z_hbm, src_hbm, o_hbm, z_v, idx_v, out_v):
        wid = lax.axis_index("s") * NUM_SC + lax.axis_index("c")
        base = wid * EDGES_PER_TILE
        pltpu.sync_copy(z_hbm, z_v)
        pltpu.sync_copy(src_hbm.at[pl.ds(base, EDGES_PER_TILE)], idx_v)

        @pl.loop(0, EDGES_PER_TILE, step=16)
        def _(i):
            idx = idx_v[pl.ds(i, 16)]
            out_v[pl.ds(i, 16)] = plsc.load_gather(z_v, [idx])

        pltpu.sync_copy(out_v, o_hbm.at[pl.ds(base, EDGES_PER_TILE)])

    return k(z1d, src1d)


def _edge_feats_body(dist_ref, zc_ref, wd_ref, tpad_ref, o_ref):
    d = dist_ref[0]
    mu = lax.broadcasted_iota(jnp.int32, (N_RBF, 1), 0).astype(jnp.float32) * (
        CUTOFF / (N_RBF - 1)
    )
    inv_sigma = (N_RBF - 1) / CUTOFF
    t = (d - mu) * inv_sigma
    rbf = jnp.exp(-0.5 * t * t)
    h = d * (jnp.pi / (2.0 * CUTOFF))
    y = h * h
    s = h * (
        1.0
        + y
        * (
            -1.0 / 6.0
            + y * (1.0 / 120.0 + y * (-1.0 / 5040.0 + y * (1.0 / 362880.0)))
        )
    )
    env = jnp.where(d < CUTOFF, 1.0 - s * s, 0.0)
    rbf_env = rbf * env
    dist_emb = lax.dot_general(
        rbf_env,
        wd_ref[...],
        (((0,), (0,)), ((), ())),
        preferred_element_type=jnp.float32,
    )
    zl = zc_ref[0]
    sub = lax.broadcasted_iota(jnp.int32, (FEAT, 1), 0)
    oh = (sub == zl).astype(jnp.float32)
    nbr = lax.dot_general(
        oh,
        tpad_ref[...],
        (((0,), (0,)), ((), ())),
        preferred_element_type=jnp.float32,
    )
    o_ref[...] = dist_emb * nbr


def _edge_feats(dist3d, zc3d, w_dist, t_pad):
    return pl.pallas_call(
        _edge_feats_body,
        grid=(NEB,),
        in_specs=[
            pl.BlockSpec((1, 1, EBL), lambda i: (i, 0, 0)),
            pl.BlockSpec((1, 1, EBL), lambda i: (i, 0, 0)),
            pl.BlockSpec((N_RBF, FEAT), lambda i: (0, 0)),
            pl.BlockSpec((FEAT, FEAT), lambda i: (0, 0)),
        ],
        out_specs=pl.BlockSpec((EBL, FEAT), lambda i: (i, 0)),
        out_shape=jax.ShapeDtypeStruct((N_EDGES, FEAT), jnp.float32),
    )(dist3d, zc3d, w_dist, t_pad)


def _scatter_add(edge_feats, dst2d, zeros_tile):
    @pl.kernel(
        out_type=jax.ShapeDtypeStruct((NUM_SC, N_PAD, FEAT), jnp.float32),
        mesh=_sc_mesh,
        scratch_types=[pltpu.VMEM_SHARED((N_PAD, FEAT), jnp.float32)],
    )
    def k(ef_hbm, dst_hbm, zeros_hbm, o_hbm, acc):
        c = lax.axis_index("c")
        s = lax.axis_index("s")
        row0 = s * ROWS_PER_SUBCORE
        pltpu.sync_copy(zeros_hbm, acc.at[pl.ds(row0, ROWS_PER_SUBCORE)])
        plsc.subcore_barrier()

        def body(x_vmem, i_vmem):
            pltpu.sync_copy(x_vmem, acc.at[i_vmem.at[0]], add=True)

        pltpu.emit_pipeline(
            body,
            grid=(N_EDGES // EDGE_WIN,),
            in_specs=[
                pl.BlockSpec((EDGE_WIN, FEAT), lambda i: (i, 0)),
                pl.BlockSpec((1, EDGE_WIN), lambda i: (0, i)),
            ],
            out_specs=[],
            core_axis_name=("c", "s"),
            dimension_semantics=(pltpu.PARALLEL,),
        )(ef_hbm, dst_hbm)
        plsc.subcore_barrier()
        pltpu.sync_copy(
            acc.at[pl.ds(row0, ROWS_PER_SUBCORE)],
            o_hbm.at[c].at[pl.ds(row0, ROWS_PER_SUBCORE)],
        )

    return k(edge_feats, dst2d, zeros_tile)


def _final_body(z_ref, parts_ref, tpad_ref, wcat_ref, b_ref, o_ref):
    lane = lax.broadcasted_iota(jnp.int32, (NB, FEAT), 1)
    oh = (z_ref[...] == lane).astype(jnp.float32)
    tw1 = jnp.dot(
        tpad_ref[...], wcat_ref[0:FEAT, :], preferred_element_type=jnp.float32
    )
    aggr = parts_ref[0] + parts_ref[1]
    o_ref[...] = (
        jnp.dot(oh, tw1, preferred_element_type=jnp.float32)
        + jnp.dot(
            aggr,
            wcat_ref[FEAT : 2 * FEAT, :],
            preferred_element_type=jnp.float32,
        )
        + b_ref[...]
    )


def _final(z2d, parts, t_pad, w_cat, b2d):
    return pl.pallas_call(
        _final_body,
        grid=(N_ATOMS // NB,),
        in_specs=[
            pl.BlockSpec((NB, 1), lambda i: (i, 0)),
            pl.BlockSpec((NUM_SC, NB, FEAT), lambda i: (0, i, 0)),
            pl.BlockSpec((FEAT, FEAT), lambda i: (0, 0)),
            pl.BlockSpec((2 * FEAT, FEAT), lambda i: (0, 0)),
            pl.BlockSpec((1, FEAT), lambda i: (0, 0)),
        ],
        out_specs=pl.BlockSpec((NB, FEAT), lambda i: (i, 0)),
        out_shape=jax.ShapeDtypeStruct((N_ATOMS, FEAT), jnp.float32),
    )(z2d, parts, t_pad, w_cat, b2d)


def kernel(z_number, nbrs, dist, atom_table, W_dist, W_cat, b_cat):
    z = z_number.astype(jnp.int32)
    dst2d = nbrs[:, 0].astype(jnp.int32).reshape(1, N_EDGES)
    src1d = nbrs[:, 1].astype(jnp.int32)
    t_pad = jnp.zeros((FEAT, FEAT), jnp.float32).at[:100].set(atom_table)

    zc1d = _zc_gather(z, src1d)
    dist3d = dist.reshape(NEB, 1, EBL)
    zc3d = zc1d.reshape(NEB, 1, EBL)
    edge_feats = _edge_feats(dist3d, zc3d, W_dist, t_pad)
    zeros_tile = jnp.zeros((ROWS_PER_SUBCORE, FEAT), jnp.float32)
    parts = _scatter_add(edge_feats, dst2d, zeros_tile)
    out = _final(
        z.reshape(N_ATOMS, 1), parts, t_pad, W_cat, b_cat.reshape(1, FEAT)
    )
    return out


# speedup vs baseline: 14.1727x; 1.1258x over previous
import jax
import jax.numpy as jnp
from jax import lax
from jax.experimental import pallas as pl
from jax.experimental.pallas import tpu as pltpu
from jax.experimental.pallas import tpu_sc as plsc

N_ATOMS = 10000
N_EDGES = 320000
FEAT = 128
N_RBF = 64
CUTOFF = 5.0

NUM_SC = 2
NUM_SUBCORES = 16
NUM_TILES = NUM_SC * NUM_SUBCORES
EDGES_PER_TILE = N_EDGES // NUM_TILES
N_PAD = 10240
ROWS_PER_SUBCORE = N_PAD // NUM_SUBCORES
EDGE_WIN = 128

EBL = 2560
NEB = N_EDGES // EBL
CHUNK_BLOCKS = (63, 62)
NB = 2000

_sc_mesh = plsc.VectorSubcoreMesh(core_axis_name="c", subcore_axis_name="s")


def _zc_gather(z1d, src1d):
    @pl.kernel(
        out_type=jax.ShapeDtypeStruct((N_EDGES,), jnp.int32),
        mesh=_sc_mesh,
        compiler_params=pltpu.CompilerParams(needs_layout_passes=False),
        scratch_types=[
            pltpu.VMEM((N_ATOMS,), jnp.int32),
            pltpu.VMEM((EDGES_PER_TILE,), jnp.int32),
            pltpu.VMEM((EDGES_PER_TILE,), jnp.int32),
        ],
    )
    def k(z_hbm, src_hbm, o_hbm, z_v, idx_v, out_v):
        wid = lax.axis_index("s") * NUM_SC + lax.axis_index("c")
        base = wid * EDGES_PER_TILE
        pltpu.sync_copy(z_hbm, z_v)
        pltpu.sync_copy(src_hbm.at[pl.ds(base, EDGES_PER_TILE)], idx_v)

        @pl.loop(0, EDGES_PER_TILE, step=16)
        def _(i):
            idx = idx_v[pl.ds(i, 16)]
            out_v[pl.ds(i, 16)] = plsc.load_gather(z_v, [idx])

        pltpu.sync_copy(out_v, o_hbm.at[pl.ds(base, EDGES_PER_TILE)])

    return k(z1d, src1d)


def _edge_feats_body(dist_ref, zc_ref, wd_ref, tpad_ref, o_ref):
    d = dist_ref[0]
    mu = lax.broadcasted_iota(jnp.int32, (N_RBF, 1), 0).astype(jnp.float32) * (
        CUTOFF / (N_RBF - 1)
    )
    inv_sigma = (N_RBF - 1) / CUTOFF
    t = (d - mu) * inv_sigma
    rbf = jnp.exp(-0.5 * t * t)
    h = d * (jnp.pi / (2.0 * CUTOFF))
    y = h * h
    s = h * (
        1.0
        + y
        * (
            -1.0 / 6.0
            + y * (1.0 / 120.0 + y * (-1.0 / 5040.0 + y * (1.0 / 362880.0)))
        )
    )
    env = jnp.where(d < CUTOFF, 1.0 - s * s, 0.0)
    rbf_env = (rbf * env).astype(jnp.bfloat16)
    dist_emb = lax.dot_general(
        rbf_env,
        wd_ref[...],
        (((0,), (0,)), ((), ())),
        preferred_element_type=jnp.float32,
    )
    zl = zc_ref[0]
    sub = lax.broadcasted_iota(jnp.int32, (FEAT, 1), 0)
    oh = (sub == zl).astype(jnp.bfloat16)
    nbr = lax.dot_general(
        oh,
        tpad_ref[...],
        (((0,), (0,)), ((), ())),
        preferred_element_type=jnp.float32,
    )
    o_ref[...] = dist_emb * nbr


def _edge_feats(dist3d, zc3d, w_dist_bf, t_pad_bf, block0, nblocks):
    return pl.pallas_call(
        _edge_feats_body,
        grid=(nblocks,),
        in_specs=[
            pl.BlockSpec((1, 1, EBL), lambda i: (i + block0, 0, 0)),
            pl.BlockSpec((1, 1, EBL), lambda i: (i + block0, 0, 0)),
            pl.BlockSpec((N_RBF, FEAT), lambda i: (0, 0)),
            pl.BlockSpec((FEAT, FEAT), lambda i: (0, 0)),
        ],
        out_specs=pl.BlockSpec((EBL, FEAT), lambda i: (i, 0)),
        out_shape=jax.ShapeDtypeStruct((nblocks * EBL, FEAT), jnp.float32),
    )(dist3d, zc3d, w_dist_bf, t_pad_bf)


def _scatter_add(ef_chunk, dst2d, zeros_tile, win0, nwin):
    @pl.kernel(
        out_type=jax.ShapeDtypeStruct((NUM_SC, N_PAD, FEAT), jnp.float32),
        mesh=_sc_mesh,
        scratch_types=[pltpu.VMEM_SHARED((N_PAD, FEAT), jnp.float32)],
    )
    def k(ef_hbm, dst_hbm, zeros_hbm, o_hbm, acc):
        c = lax.axis_index("c")
        s = lax.axis_index("s")
        row0 = s * ROWS_PER_SUBCORE
        pltpu.sync_copy(zeros_hbm, acc.at[pl.ds(row0, ROWS_PER_SUBCORE)])
        plsc.subcore_barrier()

        def body(x_vmem, i_vmem):
            pltpu.sync_copy(x_vmem, acc.at[i_vmem.at[0]], add=True)

        pltpu.emit_pipeline(
            body,
            grid=(nwin,),
            in_specs=[
                pl.BlockSpec((EDGE_WIN, FEAT), lambda i: (i, 0)),
                pl.BlockSpec((1, EDGE_WIN), lambda i: (0, i + win0)),
            ],
            out_specs=[],
            core_axis_name=("c", "s"),
            dimension_semantics=(pltpu.PARALLEL,),
        )(ef_hbm, dst_hbm)
        plsc.subcore_barrier()
        pltpu.sync_copy(
            acc.at[pl.ds(row0, ROWS_PER_SUBCORE)],
            o_hbm.at[c].at[pl.ds(row0, ROWS_PER_SUBCORE)],
        )

    return k(ef_chunk, dst2d, zeros_tile)


def _final_body(z_ref, p0_ref, p1_ref, tpad_ref, wcat_ref, b_ref, o_ref):
    lane = lax.broadcasted_iota(jnp.int32, (NB, FEAT), 1)
    oh = (z_ref[...] == lane).astype(jnp.float32)
    tw1 = jnp.dot(
        tpad_ref[...], wcat_ref[0:FEAT, :], preferred_element_type=jnp.float32
    )
    aggr = p0_ref[0] + p0_ref[1] + p1_ref[0] + p1_ref[1]
    o_ref[...] = (
        jnp.dot(oh, tw1, preferred_element_type=jnp.float32)
        + jnp.dot(
            aggr,
            wcat_ref[FEAT : 2 * FEAT, :],
            preferred_element_type=jnp.float32,
        )
        + b_ref[...]
    )


def _final(z2d, p0, p1, t_pad, w_cat, b2d):
    parts_spec = pl.BlockSpec((NUM_SC, NB, FEAT), lambda i: (0, i, 0))
    return pl.pallas_call(
        _final_body,
        grid=(N_ATOMS // NB,),
        in_specs=[
            pl.BlockSpec((NB, 1), lambda i: (i, 0)),
            parts_spec,
            parts_spec,
            pl.BlockSpec((FEAT, FEAT), lambda i: (0, 0)),
            pl.BlockSpec((2 * FEAT, FEAT), lambda i: (0, 0)),
            pl.BlockSpec((1, FEAT), lambda i: (0, 0)),
        ],
        out_specs=pl.BlockSpec((NB, FEAT), lambda i: (i, 0)),
        out_shape=jax.ShapeDtypeStruct((N_ATOMS, FEAT), jnp.float32),
    )(z2d, p0, p1, t_pad, w_cat, b2d)


def kernel(z_number, nbrs, dist, atom_table, W_dist, W_cat, b_cat):
    z = z_number.astype(jnp.int32)
    dst2d = nbrs[:, 0].astype(jnp.int32).reshape(1, N_EDGES)
    src1d = nbrs[:, 1].astype(jnp.int32)
    t_pad = jnp.zeros((FEAT, FEAT), jnp.float32).at[:100].set(atom_table)
    t_pad_bf = t_pad.astype(jnp.bfloat16)
    w_dist_bf = W_dist.astype(jnp.bfloat16)

    zc1d = _zc_gather(z, src1d)
    dist3d = dist.reshape(NEB, 1, EBL)
    zc3d = zc1d.reshape(NEB, 1, EBL)
    zeros_tile = jnp.zeros((ROWS_PER_SUBCORE, FEAT), jnp.float32)

    parts = []
    block0 = 0
    for nblocks in CHUNK_BLOCKS:
        ef = _edge_feats(dist3d, zc3d, w_dist_bf, t_pad_bf, block0, nblocks)
        win0 = block0 * (EBL // EDGE_WIN)
        nwin = nblocks * (EBL // EDGE_WIN)
        parts.append(_scatter_add(ef, dst2d, zeros_tile, win0, nwin))
        block0 += nblocks

    out = _final(
        z.reshape(N_ATOMS, 1),
        parts[0],
        parts[1],
        t_pad,
        W_cat,
        b_cat.reshape(1, FEAT),
    )
    return out


# speedup vs baseline: 14.6585x; 1.0343x over previous
import jax
import jax.numpy as jnp
from jax import lax
from jax.experimental import pallas as pl
from jax.experimental.pallas import tpu as pltpu
from jax.experimental.pallas import tpu_sc as plsc

N_ATOMS = 10000
N_EDGES = 320000
FEAT = 128
N_RBF = 64
CUTOFF = 5.0

NUM_SC = 2
NUM_SUBCORES = 16
NUM_TILES = NUM_SC * NUM_SUBCORES
EDGES_PER_TILE = N_EDGES // NUM_TILES
N_PAD = 10240
ROWS_PER_SUBCORE = N_PAD // NUM_SUBCORES
EDGE_WIN = 128

EBL = 3200
NEB = N_EDGES // EBL
CHUNK_BLOCKS = (50, 50)
WPS = 1
NB = 2000

_sc_mesh = plsc.VectorSubcoreMesh(core_axis_name="c", subcore_axis_name="s")


def _zc_gather(z1d, src1d):
    @pl.kernel(
        out_type=jax.ShapeDtypeStruct((N_EDGES,), jnp.int32),
        mesh=_sc_mesh,
        compiler_params=pltpu.CompilerParams(needs_layout_passes=False),
        scratch_types=[
            pltpu.VMEM((N_ATOMS,), jnp.int32),
            pltpu.VMEM((EDGES_PER_TILE,), jnp.int32),
            pltpu.VMEM((EDGES_PER_TILE,), jnp.int32),
        ],
    )
    def k(z_hbm, src_hbm, o_hbm, z_v, idx_v, out_v):
        wid = lax.axis_index("s") * NUM_SC + lax.axis_index("c")
        base = wid * EDGES_PER_TILE
        pltpu.sync_copy(z_hbm, z_v)
        pltpu.sync_copy(src_hbm.at[pl.ds(base, EDGES_PER_TILE)], idx_v)

        @pl.loop(0, EDGES_PER_TILE, step=16)
        def _(i):
            idx = idx_v[pl.ds(i, 16)]
            out_v[pl.ds(i, 16)] = plsc.load_gather(z_v, [idx])

        pltpu.sync_copy(out_v, o_hbm.at[pl.ds(base, EDGES_PER_TILE)])

    return k(z1d, src1d)


def _edge_feats_body(dist_ref, zc_ref, wd_ref, tpad_ref, o_ref):
    d = dist_ref[0]
    mu = lax.broadcasted_iota(jnp.int32, (N_RBF, 1), 0).astype(jnp.float32) * (
        CUTOFF / (N_RBF - 1)
    )
    inv_sigma = (N_RBF - 1) / CUTOFF
    t = (d - mu) * inv_sigma
    rbf = jnp.exp(-0.5 * t * t)
    h = d * (jnp.pi / (2.0 * CUTOFF))
    y = h * h
    s = h * (
        1.0
        + y
        * (
            -1.0 / 6.0
            + y * (1.0 / 120.0 + y * (-1.0 / 5040.0 + y * (1.0 / 362880.0)))
        )
    )
    env = jnp.where(d < CUTOFF, 1.0 - s * s, 0.0)
    rbf_env = (rbf * env).astype(jnp.bfloat16)
    dist_emb = lax.dot_general(
        rbf_env,
        wd_ref[...],
        (((0,), (0,)), ((), ())),
        preferred_element_type=jnp.float32,
    )
    zl = zc_ref[0]
    sub = lax.broadcasted_iota(jnp.int32, (FEAT, 1), 0)
    oh = (sub == zl).astype(jnp.bfloat16)
    nbr = lax.dot_general(
        oh,
        tpad_ref[...],
        (((0,), (0,)), ((), ())),
        preferred_element_type=jnp.float32,
    )
    o_ref[...] = dist_emb * nbr


def _edge_feats(dist3d, zc3d, w_dist_bf, t_pad_bf, block0, nblocks):
    return pl.pallas_call(
        _edge_feats_body,
        grid=(nblocks,),
        in_specs=[
            pl.BlockSpec((1, 1, EBL), lambda i: (i + block0, 0, 0)),
            pl.BlockSpec((1, 1, EBL), lambda i: (i + block0, 0, 0)),
            pl.BlockSpec((N_RBF, FEAT), lambda i: (0, 0)),
            pl.BlockSpec((FEAT, FEAT), lambda i: (0, 0)),
        ],
        out_specs=pl.BlockSpec((EBL, FEAT), lambda i: (i, 0)),
        out_shape=jax.ShapeDtypeStruct((nblocks * EBL, FEAT), jnp.float32),
    )(dist3d, zc3d, w_dist_bf, t_pad_bf)


def _scatter_add(ef_chunk, dst3d, init_parts, step0, nsteps):
    @pl.kernel(
        out_type=jax.ShapeDtypeStruct((NUM_SC, N_PAD, FEAT), jnp.float32),
        mesh=_sc_mesh,
        scratch_types=[pltpu.VMEM_SHARED((N_PAD, FEAT), jnp.float32)],
    )
    def k(ef_hbm, dst_hbm, init_hbm, o_hbm, acc):
        c = lax.axis_index("c")
        s = lax.axis_index("s")
        row0 = s * ROWS_PER_SUBCORE
        pltpu.sync_copy(
            init_hbm.at[c].at[pl.ds(row0, ROWS_PER_SUBCORE)],
            acc.at[pl.ds(row0, ROWS_PER_SUBCORE)],
        )
        plsc.subcore_barrier()

        def body(x_vmem, i_vmem):
            pltpu.sync_copy(x_vmem, acc.at[i_vmem.at[0, 0]], add=True)

        pltpu.emit_pipeline(
            body,
            grid=(nsteps,),
            in_specs=[
                pl.BlockSpec((WPS * EDGE_WIN, FEAT), lambda i: (i, 0)),
                pl.BlockSpec((1, 1, WPS * EDGE_WIN), lambda i: (i + step0, 0, 0)),
            ],
            out_specs=[],
            core_axis_name=("c", "s"),
            dimension_semantics=(pltpu.PARALLEL,),
        )(ef_hbm, dst_hbm)
        plsc.subcore_barrier()
        pltpu.sync_copy(
            acc.at[pl.ds(row0, ROWS_PER_SUBCORE)],
            o_hbm.at[c].at[pl.ds(row0, ROWS_PER_SUBCORE)],
        )

    return k(ef_chunk, dst3d, init_parts)


def _final_body(z_ref, p_ref, tpad_ref, wcat_ref, b_ref, o_ref):
    lane = lax.broadcasted_iota(jnp.int32, (NB, FEAT), 1)
    oh = (z_ref[...] == lane).astype(jnp.float32)
    tw1 = jnp.dot(
        tpad_ref[...], wcat_ref[0:FEAT, :], preferred_element_type=jnp.float32
    )
    aggr = p_ref[0] + p_ref[1]
    o_ref[...] = (
        jnp.dot(oh, tw1, preferred_element_type=jnp.float32)
        + jnp.dot(
            aggr,
            wcat_ref[FEAT : 2 * FEAT, :],
            preferred_element_type=jnp.float32,
        )
        + b_ref[...]
    )


def _final(z2d, p, t_pad, w_cat, b2d):
    return pl.pallas_call(
        _final_body,
        grid=(N_ATOMS // NB,),
        in_specs=[
            pl.BlockSpec((NB, 1), lambda i: (i, 0)),
            pl.BlockSpec((NUM_SC, NB, FEAT), lambda i: (0, i, 0)),
            pl.BlockSpec((FEAT, FEAT), lambda i: (0, 0)),
            pl.BlockSpec((2 * FEAT, FEAT), lambda i: (0, 0)),
            pl.BlockSpec((1, FEAT), lambda i: (0, 0)),
        ],
        out_specs=pl.BlockSpec((NB, FEAT), lambda i: (i, 0)),
        out_shape=jax.ShapeDtypeStruct((N_ATOMS, FEAT), jnp.float32),
    )(z2d, p, t_pad, w_cat, b2d)


def kernel(z_number, nbrs, dist, atom_table, W_dist, W_cat, b_cat):
    z = z_number.astype(jnp.int32)
    dst3d = nbrs[:, 0].astype(jnp.int32).reshape(
        N_EDGES // (WPS * EDGE_WIN), 1, WPS * EDGE_WIN
    )
    src1d = nbrs[:, 1].astype(jnp.int32)
    t_pad = jnp.zeros((FEAT, FEAT), jnp.float32).at[:100].set(atom_table)
    t_pad_bf = t_pad.astype(jnp.bfloat16)
    w_dist_bf = W_dist.astype(jnp.bfloat16)

    zc1d = _zc_gather(z, src1d)
    dist3d = dist.reshape(NEB, 1, EBL)
    zc3d = zc1d.reshape(NEB, 1, EBL)

    part = jnp.zeros((NUM_SC, N_PAD, FEAT), jnp.float32)
    block0 = 0
    for nblocks in CHUNK_BLOCKS:
        ef = _edge_feats(dist3d, zc3d, w_dist_bf, t_pad_bf, block0, nblocks)
        step0 = block0 * (EBL // (WPS * EDGE_WIN))
        nsteps = nblocks * (EBL // (WPS * EDGE_WIN))
        part = _scatter_add(ef, dst3d, part, step0, nsteps)
        block0 += nblocks

    out = _final(
        z.reshape(N_ATOMS, 1),
        part,
        t_pad,
        W_cat,
        b_cat.reshape(1, FEAT),
    )
    return out
